# baseline (speedup 1.0000x reference)
import functools

import jax
import jax.numpy as jnp
from jax import lax
from jax.experimental import pallas as pl
from jax.experimental.pallas import tpu as pltpu
from jax.experimental.pallas import tpu_sc as plsc

_N = 10000
_C = 128
_E = 320000

_NC = 2
_NS = 16
_EPC = _E // _NC
_EPT = _EPC // _NS
_CH = 80
_NCHUNK = _EPT // _CH
_NPAD = 10240
_RPT = _NPAD // _NS

_NB = 2000
_NBLK = _N // _NB


def _pre_body(x_ref, pos_ref, hw1, hb1, hw2, hb2, fwp, fwx, fb,
              u_ref, v_ref):
    x = x_ref[...]
    pos = pos_ref[...]
    t = jnp.maximum(x @ hw1[...] + hb1[...], 0.0)
    delta = jnp.tanh(t @ hw2[...] + hb2[...])
    pfw = pos @ fwp[...]
    u_ref[...] = x @ fwx[...] + pfw
    v_ref[...] = (delta @ fwp[...] - pfw) + fb[...]


def _pre_call(x, pos8, hw1, hb1, hw2, hb2, fwp, fwx, fb):
    full = lambda shape: pl.BlockSpec(shape, lambda i: (0, 0))
    return pl.pallas_call(
        _pre_body,
        grid=(_NBLK,),
        in_specs=[
            pl.BlockSpec((_NB, _C), lambda i: (i, 0)),
            pl.BlockSpec((_NB, 8), lambda i: (i, 0)),
            full((_C, _C)), full((1, _C)),
            full((_C, 8)), full((1, 8)),
            full((8, _C)), full((_C, _C)), full((1, _C)),
        ],
        out_specs=[
            pl.BlockSpec((_NB, _C), lambda i: (i, 0)),
            pl.BlockSpec((_NB, _C), lambda i: (i, 0)),
        ],
        out_shape=[
            jax.ShapeDtypeStruct((_N, _C), jnp.float32),
            jax.ShapeDtypeStruct((_N, _C), jnp.float32),
        ],
    )(x, pos8, hw1, hb1, hw2, hb2, fwp, fwx, fb)


def _edge_body(u_hbm, v_hbm, src_hbm, dst_hbm, out_hbm,
               sidx, didx, sdidx, urows, vrows, accum,
               gsem0, gsem1, ssem0, ssem1, isem0, isem1):
    cid = lax.axis_index("c")
    sid = lax.axis_index("s")
    gsem = (gsem0, gsem1)
    ssem = (ssem0, ssem1)
    isem = (isem0, isem1)

    tile_base = cid * _EPC + sid * _EPT

    def fire_idx(c, b):
        base = tile_base + c * _CH
        pltpu.async_copy(src_hbm.at[pl.ds(base, _CH)], sidx.at[b], isem[b])
        pltpu.async_copy(dst_hbm.at[pl.ds(base, _CH)], didx.at[b], isem[b])

    def wait_idx(b):
        pltpu.make_async_copy(src_hbm.at[pl.ds(0, _CH)], sidx.at[b],
                              isem[b]).wait()
        pltpu.make_async_copy(dst_hbm.at[pl.ds(0, _CH)], didx.at[b],
                              isem[b]).wait()

    def fire_gather(b):
        pltpu.async_copy(u_hbm.at[sidx.at[b]], urows.at[b], gsem[b])
        pltpu.async_copy(v_hbm.at[didx.at[b]], vrows.at[b], gsem[b])

    def wait_gather(b):
        pltpu.make_async_copy(u_hbm.at[sidx.at[b]], urows.at[b],
                              gsem[b]).wait()
        pltpu.make_async_copy(v_hbm.at[didx.at[b]], vrows.at[b],
                              gsem[b]).wait()

    def fire_scatter(b):
        pltpu.async_copy(urows.at[b], accum.at[sdidx.at[b]], ssem[b],
                         add=True)

    def wait_scatter(b):
        pltpu.make_async_copy(urows.at[b], accum.at[sdidx.at[b]],
                              ssem[b]).wait()

    def step(b, *, first=False, last=False):
        wait_gather(b)
        for k in range(_CH // 16):
            s = pl.ds(k * 16, 16)
            sdidx[b, s] = didx[b, s]

        @pl.loop(0, _CH, unroll=4)
        def _rbody(r):
            for g in range(_C // 16):
                s = pl.ds(g * 16, 16)
                urows[b, r, s] = jnp.maximum(
                    urows[b, r, s] + vrows[b, r, s], 0.0)

        fire_scatter(b)
        if not last:
            wait_idx(1 - b)
            if not first:
                wait_scatter(1 - b)
            fire_gather(1 - b)

    zero16 = jnp.zeros((16,), jnp.float32)

    @pl.loop(0, _CH, unroll=4)
    def _zb(r):
        for g in range(_C // 16):
            vrows[0, r, pl.ds(g * 16, 16)] = zero16

    for k in range(_RPT // _CH):
        pltpu.sync_copy(vrows.at[0],
                        accum.at[pl.ds(sid * _RPT + k * _CH, _CH)])
    plsc.subcore_barrier()

    pltpu.sync_copy(src_hbm.at[pl.ds(tile_base, _CH)], sidx.at[0])
    pltpu.sync_copy(dst_hbm.at[pl.ds(tile_base, _CH)], didx.at[0])
    fire_gather(0)
    fire_idx(1, 1)

    step(0, first=True)
    fire_idx(2, 0)
    step(1)
    fire_idx(3, 1)

    @pl.loop(1, (_NCHUNK - 5) // 2 + 1)
    def _pair(t):
        step(0)
        fire_idx(2 * t + 2, 0)
        step(1)
        fire_idx(2 * t + 3, 1)

    step(0)
    fire_idx(_NCHUNK - 1, 0)
    step(1)
    step(0, last=True)
    wait_scatter(1)
    wait_scatter(0)

    plsc.subcore_barrier()
    pltpu.sync_copy(accum.at[pl.ds(sid * _RPT, _RPT)],
                    out_hbm.at[pl.ds(cid * _NPAD + sid * _RPT, _RPT)])


_edge_call = functools.partial(
    pl.kernel,
    mesh=plsc.VectorSubcoreMesh(core_axis_name="c", subcore_axis_name="s"),
    out_type=jax.ShapeDtypeStruct((_NC * _NPAD, _C), jnp.float32),
    scratch_types=[
        pltpu.VMEM((2, _CH), jnp.int32),
        pltpu.VMEM((2, _CH), jnp.int32),
        pltpu.VMEM((2, _CH), jnp.int32),
        pltpu.VMEM((2, _CH, _C), jnp.float32),
        pltpu.VMEM((2, _CH, _C), jnp.float32),
        pltpu.VMEM_SHARED((_NPAD, _C), jnp.float32),
        pltpu.SemaphoreType.DMA,
        pltpu.SemaphoreType.DMA,
        pltpu.SemaphoreType.DMA,
        pltpu.SemaphoreType.DMA,
        pltpu.SemaphoreType.DMA,
        pltpu.SemaphoreType.DMA,
    ],
)(_edge_body)


def _post_body(x_ref, p0_ref, p1_ref, gw1, gb1, gw2, gb2, o_ref):
    aggr = p0_ref[0] + p1_ref[0]
    t = jnp.maximum(aggr @ gw1[...] + gb1[...], 0.0)
    o_ref[...] = x_ref[...] + (t @ gw2[...] + gb2[...])


def _post_call(x, parts, gw1, gb1, gw2, gb2):
    full = lambda shape: pl.BlockSpec(shape, lambda i: (0, 0))
    return pl.pallas_call(
        _post_body,
        grid=(_NBLK,),
        in_specs=[
            pl.BlockSpec((_NB, _C), lambda i: (i, 0)),
            pl.BlockSpec((1, _NB, _C), lambda i: (0, i, 0)),
            pl.BlockSpec((1, _NB, _C), lambda i: (1, i, 0)),
            full((_C, _C)), full((1, _C)),
            full((_C, _C)), full((1, _C)),
        ],
        out_specs=pl.BlockSpec((_NB, _C), lambda i: (i, 0)),
        out_shape=jax.ShapeDtypeStruct((_N, _C), jnp.float32),
    )(x, parts, parts, gw1, gb1, gw2, gb2)


def kernel(x, pos, edge_index,
           h_w1, h_b1, h_w2, h_b2, f_w, f_b, g_w1, g_b1, g_w2, g_b2):
    src = edge_index[0].astype(jnp.int32)
    dst = edge_index[1].astype(jnp.int32)

    fwp = jnp.pad(f_w[:3], ((0, 5), (0, 0)))
    fwx = f_w[3:]
    pos8 = jnp.pad(pos, ((0, 0), (0, 5)))
    hw2 = jnp.pad(h_w2, ((0, 0), (0, 5)))
    hb2 = jnp.pad(h_b2, (0, 5)).reshape(1, 8)

    u, v = _pre_call(x, pos8, h_w1, h_b1.reshape(1, _C), hw2, hb2,
                     fwp, fwx, f_b.reshape(1, _C))
    parts = _edge_call(u, v, src, dst).reshape(_NC, _NPAD, _C)
    return _post_call(x, parts, g_w1, g_b1.reshape(1, _C),
                      g_w2, g_b2.reshape(1, _C))


# speedup vs baseline: 1.1316x; 1.1316x over previous
import functools

import jax
import jax.numpy as jnp
from jax import lax
from jax.experimental import pallas as pl
from jax.experimental.pallas import tpu as pltpu
from jax.experimental.pallas import tpu_sc as plsc

_N = 10000
_C = 128
_E = 320000

_NC = 2
_NS = 16
_EPC = _E // _NC
_EPT = _EPC // _NS
_CH = 80
_NCHUNK = _EPT // _CH
_NPAD = 10240
_RPT = _NPAD // _NS

_NB = 2000
_NBLK = _N // _NB


def _pre_body(x_ref, pos_ref, hw1, hb1, hw2, hb2, fwp, fwx, fb,
              u_ref, v_ref):
    x = x_ref[...]
    pos = pos_ref[...]
    t = jnp.maximum(x @ hw1[...] + hb1[...], 0.0)
    delta = jnp.tanh(t @ hw2[...] + hb2[...])
    pfw = pos @ fwp[...]
    u_ref[...] = x @ fwx[...] + pfw
    v_ref[...] = (delta @ fwp[...] - pfw) + fb[...]


def _pre_call(x, pos8, hw1, hb1, hw2, hb2, fwp, fwx, fb):
    full = lambda shape: pl.BlockSpec(shape, lambda i: (0, 0))
    return pl.pallas_call(
        _pre_body,
        grid=(_NBLK,),
        in_specs=[
            pl.BlockSpec((_NB, _C), lambda i: (i, 0)),
            pl.BlockSpec((_NB, 8), lambda i: (i, 0)),
            full((_C, _C)), full((1, _C)),
            full((_C, 8)), full((1, 8)),
            full((8, _C)), full((_C, _C)), full((1, _C)),
        ],
        out_specs=[
            pl.BlockSpec((_NB, _C), lambda i: (i, 0)),
            pl.BlockSpec((_NB, _C), lambda i: (i, 0)),
        ],
        out_shape=[
            jax.ShapeDtypeStruct((_N, _C), jnp.float32),
            jax.ShapeDtypeStruct((_N, _C), jnp.float32),
        ],
    )(x, pos8, hw1, hb1, hw2, hb2, fwp, fwx, fb)


def _edge_body(u_hbm, v_hbm, src_hbm, dst_hbm, out_hbm,
               sidx, didx, urows, vrows, accum,
               gsem0, gsem1, isem0, isem1):
    cid = lax.axis_index("c")
    sid = lax.axis_index("s")
    gsem = (gsem0, gsem1)
    isem = (isem0, isem1)

    tile_base = cid * _EPC + sid * _EPT

    def fire_idx(c, b):
        base = tile_base + c * _CH
        pltpu.async_copy(src_hbm.at[pl.ds(base, _CH)], sidx.at[b], isem[b])
        pltpu.async_copy(dst_hbm.at[pl.ds(base, _CH)], didx.at[b], isem[b])

    def wait_idx(b):
        pltpu.make_async_copy(src_hbm.at[pl.ds(0, _CH)], sidx.at[b],
                              isem[b]).wait()
        pltpu.make_async_copy(dst_hbm.at[pl.ds(0, _CH)], didx.at[b],
                              isem[b]).wait()

    def fire_gather(b):
        pltpu.async_copy(u_hbm.at[sidx.at[b]], urows.at[b], gsem[b])
        pltpu.async_copy(v_hbm.at[didx.at[b]], vrows.at[b], gsem[b])

    def wait_gather(b):
        pltpu.make_async_copy(u_hbm.at[sidx.at[b]], urows.at[b],
                              gsem[b]).wait()
        pltpu.make_async_copy(v_hbm.at[didx.at[b]], vrows.at[b],
                              gsem[b]).wait()

    def step(b, *, last=False):
        if not last:
            wait_idx(1 - b)
            fire_gather(1 - b)
        wait_gather(b)

        @pl.loop(0, _CH, unroll=4)
        def _rbody(r):
            for g in range(_C // 16):
                s = pl.ds(g * 16, 16)
                urows[b, r, s] = jnp.maximum(
                    urows[b, r, s] + vrows[b, r, s], 0.0)

        pltpu.sync_copy(urows.at[b], accum.at[didx.at[b]], add=True)

    zero16 = jnp.zeros((16,), jnp.float32)

    @pl.loop(0, _CH, unroll=4)
    def _zb(r):
        for g in range(_C // 16):
            vrows[0, r, pl.ds(g * 16, 16)] = zero16

    for k in range(_RPT // _CH):
        pltpu.sync_copy(vrows.at[0],
                        accum.at[pl.ds(sid * _RPT + k * _CH, _CH)])
    plsc.subcore_barrier()

    pltpu.sync_copy(src_hbm.at[pl.ds(tile_base, _CH)], sidx.at[0])
    pltpu.sync_copy(dst_hbm.at[pl.ds(tile_base, _CH)], didx.at[0])
    fire_gather(0)
    fire_idx(1, 1)

    step(0)
    fire_idx(2, 0)

    @pl.loop(0, (_NCHUNK - 3) // 2)
    def _pair(t):
        step(1)
        fire_idx(2 * t + 3, 1)
        step(0)
        fire_idx(2 * t + 4, 0)

    step(1)
    step(0, last=True)

    plsc.subcore_barrier()
    pltpu.sync_copy(accum.at[pl.ds(sid * _RPT, _RPT)],
                    out_hbm.at[pl.ds(cid * _NPAD + sid * _RPT, _RPT)])


_edge_call = functools.partial(
    pl.kernel,
    mesh=plsc.VectorSubcoreMesh(core_axis_name="c", subcore_axis_name="s"),
    out_type=jax.ShapeDtypeStruct((_NC * _NPAD, _C), jnp.float32),
    scratch_types=[
        pltpu.VMEM((2, _CH), jnp.int32),
        pltpu.VMEM((2, _CH), jnp.int32),
        pltpu.VMEM((2, _CH, _C), jnp.float32),
        pltpu.VMEM((2, _CH, _C), jnp.float32),
        pltpu.VMEM_SHARED((_NPAD, _C), jnp.float32),
        pltpu.SemaphoreType.DMA,
        pltpu.SemaphoreType.DMA,
        pltpu.SemaphoreType.DMA,
        pltpu.SemaphoreType.DMA,
    ],
)(_edge_body)


def _post_body(x_ref, p0_ref, p1_ref, gw1, gb1, gw2, gb2, o_ref):
    aggr = p0_ref[0] + p1_ref[0]
    t = jnp.maximum(aggr @ gw1[...] + gb1[...], 0.0)
    o_ref[...] = x_ref[...] + (t @ gw2[...] + gb2[...])


def _post_call(x, parts, gw1, gb1, gw2, gb2):
    full = lambda shape: pl.BlockSpec(shape, lambda i: (0, 0))
    return pl.pallas_call(
        _post_body,
        grid=(_NBLK,),
        in_specs=[
            pl.BlockSpec((_NB, _C), lambda i: (i, 0)),
            pl.BlockSpec((1, _NB, _C), lambda i: (0, i, 0)),
            pl.BlockSpec((1, _NB, _C), lambda i: (1, i, 0)),
            full((_C, _C)), full((1, _C)),
            full((_C, _C)), full((1, _C)),
        ],
        out_specs=pl.BlockSpec((_NB, _C), lambda i: (i, 0)),
        out_shape=jax.ShapeDtypeStruct((_N, _C), jnp.float32),
    )(x, parts, parts, gw1, gb1, gw2, gb2)


def kernel(x, pos, edge_index,
           h_w1, h_b1, h_w2, h_b2, f_w, f_b, g_w1, g_b1, g_w2, g_b2):
    src = edge_index[0].astype(jnp.int32)
    dst = edge_index[1].astype(jnp.int32)

    fwp = jnp.pad(f_w[:3], ((0, 5), (0, 0)))
    fwx = f_w[3:]
    pos8 = jnp.pad(pos, ((0, 0), (0, 5)))
    hw2 = jnp.pad(h_w2, ((0, 0), (0, 5)))
    hb2 = jnp.pad(h_b2, (0, 5)).reshape(1, 8)

    u, v = _pre_call(x, pos8, h_w1, h_b1.reshape(1, _C), hw2, hb2,
                     fwp, fwx, f_b.reshape(1, _C))
    parts = _edge_call(u, v, src, dst).reshape(_NC, _NPAD, _C)
    return _post_call(x, parts, g_w1, g_b1.reshape(1, _C),
                      g_w2, g_b2.reshape(1, _C))


# speedup vs baseline: 2.3788x; 2.1020x over previous
import functools

import jax
import jax.numpy as jnp
from jax import lax
from jax.experimental import pallas as pl
from jax.experimental.pallas import tpu as pltpu
from jax.experimental.pallas import tpu_sc as plsc

_N = 10000
_C = 128
_E = 320000

_NC = 2
_NS = 16
_EPC = _E // _NC
_EPT = _EPC // _NS
_CH = 80
_NCHUNK = _EPT // _CH
_NPAD = 10240
_RPT = _NPAD // _NS

_NB = 2000
_NBLK = _N // _NB


def _pre_body(x_ref, pos_ref, hw1, hb1, hw2, hb2, fwp, fwx, fb,
              u_ref, v_ref):
    x = x_ref[...]
    pos = pos_ref[...]
    t = jnp.maximum(x @ hw1[...] + hb1[...], 0.0)
    delta = jnp.tanh(t @ hw2[...] + hb2[...])
    pfw = pos @ fwp[...]
    u_ref[...] = x @ fwx[...] + pfw
    v_ref[...] = (delta @ fwp[...] - pfw) + fb[...]


def _pre_call(x, pos8, hw1, hb1, hw2, hb2, fwp, fwx, fb):
    full = lambda shape: pl.BlockSpec(shape, lambda i: (0, 0))
    return pl.pallas_call(
        _pre_body,
        grid=(_NBLK,),
        in_specs=[
            pl.BlockSpec((_NB, _C), lambda i: (i, 0)),
            pl.BlockSpec((_NB, 8), lambda i: (i, 0)),
            full((_C, _C)), full((1, _C)),
            full((_C, 8)), full((1, 8)),
            full((8, _C)), full((_C, _C)), full((1, _C)),
        ],
        out_specs=[
            pl.BlockSpec((_NB, _C), lambda i: (i, 0)),
            pl.BlockSpec((_NB, _C), lambda i: (i, 0)),
        ],
        out_shape=[
            jax.ShapeDtypeStruct((_N, _C), jnp.float32),
            jax.ShapeDtypeStruct((_N, _C), jnp.float32),
        ],
    )(x, pos8, hw1, hb1, hw2, hb2, fwp, fwx, fb)


def _edge_body(u_hbm, v_hbm, src_hbm, dst_hbm, out_hbm,
               sidx, didx, urows, vrows, accum,
               gsem0, gsem1, isem0, isem1):
    cid = lax.axis_index("c")
    sid = lax.axis_index("s")
    gsem = (gsem0, gsem1)
    isem = (isem0, isem1)

    tile_base = cid * _EPC + sid * _EPT

    def fire_idx(c, b):
        base = tile_base + c * _CH
        pltpu.async_copy(src_hbm.at[pl.ds(base, _CH)], sidx.at[b], isem[b])
        pltpu.async_copy(dst_hbm.at[pl.ds(base, _CH)], didx.at[b], isem[b])

    def wait_idx(b):
        pltpu.make_async_copy(src_hbm.at[pl.ds(0, _CH)], sidx.at[b],
                              isem[b]).wait()
        pltpu.make_async_copy(dst_hbm.at[pl.ds(0, _CH)], didx.at[b],
                              isem[b]).wait()

    def fire_gather(b):
        pltpu.async_copy(u_hbm.at[sidx.at[b]], urows.at[b], gsem[b])
        pltpu.async_copy(v_hbm.at[didx.at[b]], vrows.at[b], gsem[b])

    def wait_gather(b):
        pltpu.make_async_copy(u_hbm.at[sidx.at[b]], urows.at[b],
                              gsem[b]).wait()
        pltpu.make_async_copy(v_hbm.at[didx.at[b]], vrows.at[b],
                              gsem[b]).wait()

    def step(b, *, last=False):
        if not last:
            wait_idx(1 - b)
            fire_gather(1 - b)
        wait_gather(b)

        def _rbody(r, carry):
            for g in range(_C // 16):
                s = pl.ds(g * 16, 16)
                urows[b, r, s] = jnp.maximum(
                    urows[b, r, s] + vrows[b, r, s], 0.0)
            return carry

        lax.fori_loop(0, _CH, _rbody, 0)
        pltpu.sync_copy(urows.at[b], accum.at[didx.at[b]], add=True)

    zero16 = jnp.zeros((16,), jnp.float32)

    @pl.loop(0, _CH, unroll=4)
    def _zb(r):
        for g in range(_C // 16):
            vrows[0, r, pl.ds(g * 16, 16)] = zero16

    for k in range(_RPT // _CH):
        pltpu.sync_copy(vrows.at[0],
                        accum.at[pl.ds(sid * _RPT + k * _CH, _CH)])
    plsc.subcore_barrier()

    pltpu.sync_copy(src_hbm.at[pl.ds(tile_base, _CH)], sidx.at[0])
    pltpu.sync_copy(dst_hbm.at[pl.ds(tile_base, _CH)], didx.at[0])
    fire_gather(0)
    fire_idx(1, 1)

    step(0)
    fire_idx(2, 0)

    @pl.loop(0, (_NCHUNK - 3) // 2)
    def _pair(t):
        step(1)
        fire_idx(2 * t + 3, 1)
        step(0)
        fire_idx(2 * t + 4, 0)

    step(1)
    step(0, last=True)

    plsc.subcore_barrier()
    pltpu.sync_copy(accum.at[pl.ds(sid * _RPT, _RPT)],
                    out_hbm.at[pl.ds(cid * _NPAD + sid * _RPT, _RPT)])


_edge_call = functools.partial(
    pl.kernel,
    mesh=plsc.VectorSubcoreMesh(core_axis_name="c", subcore_axis_name="s"),
    out_type=jax.ShapeDtypeStruct((_NC * _NPAD, _C), jnp.float32),
    scratch_types=[
        pltpu.VMEM((2, _CH), jnp.int32),
        pltpu.VMEM((2, _CH), jnp.int32),
        pltpu.VMEM((2, _CH, _C), jnp.float32),
        pltpu.VMEM((2, _CH, _C), jnp.float32),
        pltpu.VMEM_SHARED((_NPAD, _C), jnp.float32),
        pltpu.SemaphoreType.DMA,
        pltpu.SemaphoreType.DMA,
        pltpu.SemaphoreType.DMA,
        pltpu.SemaphoreType.DMA,
    ],
)(_edge_body)


def _post_body(x_ref, p0_ref, p1_ref, gw1, gb1, gw2, gb2, o_ref):
    aggr = p0_ref[0] + p1_ref[0]
    t = jnp.maximum(aggr @ gw1[...] + gb1[...], 0.0)
    o_ref[...] = x_ref[...] + (t @ gw2[...] + gb2[...])


def _post_call(x, parts, gw1, gb1, gw2, gb2):
    full = lambda shape: pl.BlockSpec(shape, lambda i: (0, 0))
    return pl.pallas_call(
        _post_body,
        grid=(_NBLK,),
        in_specs=[
            pl.BlockSpec((_NB, _C), lambda i: (i, 0)),
            pl.BlockSpec((1, _NB, _C), lambda i: (0, i, 0)),
            pl.BlockSpec((1, _NB, _C), lambda i: (1, i, 0)),
            full((_C, _C)), full((1, _C)),
            full((_C, _C)), full((1, _C)),
        ],
        out_specs=pl.BlockSpec((_NB, _C), lambda i: (i, 0)),
        out_shape=jax.ShapeDtypeStruct((_N, _C), jnp.float32),
    )(x, parts, parts, gw1, gb1, gw2, gb2)


def kernel(x, pos, edge_index,
           h_w1, h_b1, h_w2, h_b2, f_w, f_b, g_w1, g_b1, g_w2, g_b2):
    src = edge_index[0].astype(jnp.int32)
    dst = edge_index[1].astype(jnp.int32)

    fwp = jnp.pad(f_w[:3], ((0, 5), (0, 0)))
    fwx = f_w[3:]
    pos8 = jnp.pad(pos, ((0, 0), (0, 5)))
    hw2 = jnp.pad(h_w2, ((0, 0), (0, 5)))
    hb2 = jnp.pad(h_b2, (0, 5)).reshape(1, 8)

    u, v = _pre_call(x, pos8, h_w1, h_b1.reshape(1, _C), hw2, hb2,
                     fwp, fwx, f_b.reshape(1, _C))
    parts = _edge_call(u, v, src, dst).reshape(_NC, _NPAD, _C)
    return _post_call(x, parts, g_w1, g_b1.reshape(1, _C),
                      g_w2, g_b2.reshape(1, _C))
